# Initial kernel scaffold; baseline (speedup 1.0000x reference)
import jax
import jax.numpy as jnp
from jax.experimental import pallas as pl


def kernel(tris):
    raise NotImplementedError("write your pallas kernel here")


# speedup vs baseline: 6.9605x; 6.9605x over previous
import functools

import jax
import jax.numpy as jnp
from jax import lax
from jax.experimental import pallas as pl
from jax.experimental.pallas import tpu as pltpu
from jax.experimental.pallas import tpu_sc as plsc

SIZE = 256
NTRI = 256
L = 16
NWORK = 32
ROWS_PER_W = SIZE // NWORK
NCHUNK = SIZE // L


def _bc(x):
    return jnp.full((L,), x)


def _sc_body(tris_hbm, lin_hbm, out_hbm,
             tris_v, lin_v, invw_v, degen_v, zbuf_v, tbest_v, outbuf_v):
    wid = lax.axis_index("s") * 2 + lax.axis_index("c")
    row0 = wid * ROWS_PER_W

    pltpu.sync_copy(tris_hbm, tris_v)
    pltpu.sync_copy(lin_hbm, lin_v)

    lanes = lax.iota(jnp.int32, 16)

    def meta_step(k, zmin_vec):
        tids = lanes + k * L
        base = tids * 9
        ax = plsc.load_gather(tris_v, [base + 0])
        ay = plsc.load_gather(tris_v, [base + 1])
        az = plsc.load_gather(tris_v, [base + 2])
        bx = plsc.load_gather(tris_v, [base + 3])
        by = plsc.load_gather(tris_v, [base + 4])
        bz = plsc.load_gather(tris_v, [base + 5])
        cx = plsc.load_gather(tris_v, [base + 6])
        cy = plsc.load_gather(tris_v, [base + 7])
        cz = plsc.load_gather(tris_v, [base + 8])
        w = (bx - ax) * (cy - ay) - (by - ay) * (cx - ax)
        degen = jnp.abs(w) <= 1e-8
        w_safe = jnp.where(degen, jnp.float32(1.0), w)
        invw = jnp.float32(1.0) / w_safe
        invw_v[pl.ds(k * L, L)] = invw
        degen_v[pl.ds(k * L, L)] = jnp.where(degen, jnp.float32(1.0),
                                             jnp.float32(0.0))
        zmin_vec = jnp.minimum(zmin_vec, jnp.minimum(jnp.minimum(az, bz), cz))
        return zmin_vec

    zmin_vec = lax.fori_loop(0, NTRI // L, meta_step,
                             jnp.full((L,), jnp.float32(jnp.inf)))
    zmin = jnp.min(zmin_vec)

    zinit = _bc(zmin)
    tinit = jnp.full((L,), jnp.int32(-1))

    def init_step(i, _):
        zbuf_v[pl.ds(i * L, L)] = zinit
        tbest_v[pl.ds(i * L, L)] = tinit
        return 0

    lax.fori_loop(0, (ROWS_PER_W * SIZE) // L, init_step, 0)

    def tri_step(t, _):
        base = _bc(t * 9)
        ax = plsc.load_gather(tris_v, [base + 0])
        ay = plsc.load_gather(tris_v, [base + 1])
        az = plsc.load_gather(tris_v, [base + 2])
        bx = plsc.load_gather(tris_v, [base + 3])
        by = plsc.load_gather(tris_v, [base + 4])
        bz = plsc.load_gather(tris_v, [base + 5])
        cx = plsc.load_gather(tris_v, [base + 6])
        cy = plsc.load_gather(tris_v, [base + 7])
        cz = plsc.load_gather(tris_v, [base + 8])
        ivw = plsc.load_gather(invw_v, [_bc(t)])
        dgm = plsc.load_gather(degen_v, [_bc(t)]) < 0.5
        tvec = _bc(t)

        def row_step(r, _):
            px = plsc.load_gather(lin_v, [_bc(row0 + r)])
            s1 = ax - px
            s2 = bx - px
            s3 = cx - px
            for jc in range(NCHUNK):
                off = r * SIZE + jc * L
                py = lin_v[pl.ds(jc * L, L)]
                vay = ay - py
                vby = by - py
                vcy = cy - py
                pab = s1 * vby - s2 * vay
                pbc = s2 * vcy - s3 * vby
                pca = s3 * vay - s1 * vcy
                inside = (jnp.maximum(pab, 0.0) * jnp.maximum(pbc, 0.0)
                          * jnp.maximum(pca, 0.0)) > 0
                w1 = pab * ivw
                w2 = pbc * ivw
                w3 = 1.0 - w1 - w2
                z = (w1 * az + w2 * bz) + w3 * cz
                zb = zbuf_v[pl.ds(off, L)]
                m = inside & (z >= zb) & dgm
                zbuf_v[pl.ds(off, L)] = jnp.where(m, z, zb)
                tb = tbest_v[pl.ds(off, L)]
                tbest_v[pl.ds(off, L)] = jnp.where(m, tvec, tb)
            return 0

        lax.fori_loop(0, ROWS_PER_W, row_step, 0)
        return 0

    lax.fori_loop(0, NTRI, tri_step, 0)

    def res_row(r, _):
        px = plsc.load_gather(lin_v, [_bc(row0 + r)])
        for jc in range(NCHUNK):
            off = r * SIZE + jc * L
            tb = tbest_v[pl.ds(off, L)]
            m = tb >= 0
            ti = jnp.maximum(tb, 0)
            base = ti * 9
            ax = plsc.load_gather(tris_v, [base + 0])
            ay = plsc.load_gather(tris_v, [base + 1])
            az = plsc.load_gather(tris_v, [base + 2])
            bx = plsc.load_gather(tris_v, [base + 3])
            by = plsc.load_gather(tris_v, [base + 4])
            bz = plsc.load_gather(tris_v, [base + 5])
            cx = plsc.load_gather(tris_v, [base + 6])
            cy = plsc.load_gather(tris_v, [base + 7])
            cz = plsc.load_gather(tris_v, [base + 8])
            ivw = plsc.load_gather(invw_v, [ti])
            py = lin_v[pl.ds(jc * L, L)]
            s1 = ax - px
            s2 = bx - px
            s3 = cx - px
            vay = ay - py
            vby = by - py
            vcy = cy - py
            pab = s1 * vby - s2 * vay
            pbc = s2 * vcy - s3 * vby
            w1 = pab * ivw
            w2 = pbc * ivw
            w3 = 1.0 - w1 - w2
            rr = (w1 * ax + w2 * bx) + w3 * cx
            gg = (w1 * ay + w2 * by) + w3 * cy
            bb = (w1 * az + w2 * bz) + w3 * cz
            zero = jnp.zeros((L,), jnp.float32)
            rr = jnp.where(m, rr, zero)
            gg = jnp.where(m, gg, zero)
            bb = jnp.where(m, bb, zero)
            aa = jnp.where(m, jnp.float32(1.0), zero)
            obase = r * (SIZE * 4) + jc * (L * 4)
            idx4 = lanes * 4 + obase
            plsc.store_scatter(outbuf_v, [idx4], rr)
            plsc.store_scatter(outbuf_v, [idx4 + 1], gg)
            plsc.store_scatter(outbuf_v, [idx4 + 2], bb)
            plsc.store_scatter(outbuf_v, [idx4 + 3], aa)
        return 0

    lax.fori_loop(0, ROWS_PER_W, res_row, 0)

    pltpu.sync_copy(outbuf_v, out_hbm.at[pl.ds(row0 * SIZE * 4,
                                               ROWS_PER_W * SIZE * 4)])


@jax.jit
def kernel(tris):
    tris_flat = tris.reshape(-1).astype(jnp.float32)
    lin = jnp.linspace(-1.0, 1.0, SIZE, dtype=jnp.float32)
    mesh = plsc.VectorSubcoreMesh(core_axis_name="c", subcore_axis_name="s")
    out = pl.kernel(
        _sc_body,
        out_type=jax.ShapeDtypeStruct((SIZE * SIZE * 4,), jnp.float32),
        mesh=mesh,
        compiler_params=pltpu.CompilerParams(needs_layout_passes=False),
        scratch_types=[
            pltpu.VMEM((NTRI * 9,), jnp.float32),
            pltpu.VMEM((SIZE,), jnp.float32),
            pltpu.VMEM((NTRI,), jnp.float32),
            pltpu.VMEM((NTRI,), jnp.float32),
            pltpu.VMEM((ROWS_PER_W * SIZE,), jnp.float32),
            pltpu.VMEM((ROWS_PER_W * SIZE,), jnp.int32),
            pltpu.VMEM((ROWS_PER_W * SIZE * 4,), jnp.float32),
        ],
    )(tris_flat, lin)
    return out.reshape(SIZE, SIZE, 4)
